# baseline (speedup 1.0000x reference)
import functools

import jax
import jax.numpy as jnp
from jax import lax
from jax.experimental import pallas as pl
from jax.experimental.pallas import tpu as pltpu
from jax.experimental.pallas import tpu_sc as plsc

_CHANS = 64
_SAMPLES = 1024
_DIM = 128
_BATCH = 4
_NC = 2
_NS = 16
_NW = _NC * _NS
_SPW = (_BATCH * _SAMPLES) // _NW
_SCB = 4
_NSLAB = _SPW // _SCB


def _sc_body(xt_hbm, t_hbm, cb_hbm, w_hbm, out_hbm, x_v, t_v, cb_v, w_v, out_v, sem):
    wid = lax.axis_index("s") * _NC + lax.axis_index("c")
    b_idx = wid // (_SAMPLES // _SPW)
    s0 = (wid % (_SAMPLES // _SPW)) * _SPW

    pltpu.sync_copy(xt_hbm.at[b_idx, pl.ds(s0 * _CHANS, _SPW * _CHANS)], x_v)
    pltpu.sync_copy(t_hbm.at[pl.ds(s0, _SPW), :], t_v)
    pltpu.sync_copy(cb_hbm, cb_v)
    pltpu.sync_copy(w_hbm, w_v)

    w_regs = [w_v[pl.ds(j * 16, 16)] for j in range(_DIM // 16)]

    def slab_body(g, carry):
        def sample_body(sl, c2):
            s_loc = g * _SCB + sl
            tb = [t_v[s_loc, pl.ds(j * 16, 16)] for j in range(_DIM // 16)]
            for c16 in range(_CHANS // 16):
                xrow = x_v[pl.ds(s_loc * _CHANS + c16 * 16, 16)]
                for ci in range(16):
                    xv = jnp.full((16,), xrow[ci], dtype=jnp.float32)
                    tok = sl * _CHANS + c16 * 16 + ci
                    c = c16 * 16 + ci
                    for j in range(_DIM // 16):
                        val = xv * w_regs[j] + (tb[j] + cb_v[c, pl.ds(j * 16, 16)])
                        out_v[tok, pl.ds(j * 16, 16)] = val
            return c2

        lax.fori_loop(0, _SCB, sample_body, 0)
        tok0 = s0 * _CHANS + g * (_SCB * _CHANS)
        pltpu.sync_copy(out_v, out_hbm.at[b_idx, pl.ds(tok0, _SCB * _CHANS), :])
        return carry

    lax.fori_loop(0, _NSLAB, slab_body, 0)


@functools.partial(jax.jit, static_argnames=())
def kernel(x, t_table, c_table, W, b):
    xt = jnp.transpose(x[:, 0], (0, 2, 1)).reshape(_BATCH, _SAMPLES * _CHANS)
    cb = c_table + b[None, :]
    wv = W[:, 0]
    mesh = plsc.VectorSubcoreMesh(
        core_axis_name="c", subcore_axis_name="s",
        num_cores=_NC, num_subcores=_NS)
    f = pl.kernel(
        _sc_body,
        out_type=jax.ShapeDtypeStruct((_BATCH, _SAMPLES * _CHANS, _DIM), jnp.float32),
        mesh=mesh,
        scratch_types=[
            pltpu.VMEM((_SPW * _CHANS,), jnp.float32),
            pltpu.VMEM((_SPW, _DIM), jnp.float32),
            pltpu.VMEM((_CHANS, _DIM), jnp.float32),
            pltpu.VMEM((_DIM,), jnp.float32),
            pltpu.VMEM((_SCB * _CHANS, _DIM), jnp.float32),
            pltpu.SemaphoreType.DMA,
        ],
    )
    return f(xt, t_table, cb, wv)
